# baseline (speedup 1.0000x reference)
import jax
import jax.numpy as jnp
from jax import lax
from jax.experimental import pallas as pl
from jax.experimental.pallas import tpu as pltpu
from jax.experimental.pallas import tpu_sc as plsc

C = 128
V = 100000
K = 6
CH = 56
NW = 32
V_PAD = ((V + CH * NW - 1) // (CH * NW)) * (CH * NW)
NCHUNK = V_PAD // CH
NCH_W = NCHUNK // NW
V_W = NCH_W * CH

VB1 = 4096
NB1 = (V + VB1 - 1) // VB1
VB2 = 14336
NB2 = V_PAD // VB2
EPS = 1e-5


def _mm_body(fet_ref, wn_ref, t_ref):
    fetb = fet_ref[...]
    for k in range(K):
        t_ref[k] = lax.dot_general(
            fetb, wn_ref[k], (((1,), (0,)), ((), ())),
            preferred_element_type=jnp.float32)


def _mm_call(fet, wn):
    return pl.pallas_call(
        _mm_body,
        grid=(NB1,),
        in_specs=[
            pl.BlockSpec((VB1, C), lambda j: (j, 0)),
            pl.BlockSpec((K, C, C), lambda j: (0, 0, 0)),
        ],
        out_specs=pl.BlockSpec((K, VB1, C), lambda j: (0, j, 0)),
        out_shape=jax.ShapeDtypeStruct((K, V, C), jnp.float32),
    )(fet, wn)


def _sc_body(t2, idxr, s_out, slab, gb0, gb1, ac0, ac1,
             gs0, gs1, os0, os1):
    gbuf = (gb0, gb1)
    acc = (ac0, ac1)
    gsem = (gs0, gs1)
    osem = (os0, os1)
    wid = lax.axis_index("s") * 2 + lax.axis_index("c")
    wbase = wid * V_W

    slab_cps = [pltpu.make_async_copy(
        idxr.at[pl.ds(k * V_PAD + wbase, V_W)],
        slab.at[pl.ds(k * V_W, V_W)],
        osem[0]) for k in range(K)]
    for cp in slab_cps:
        cp.start()
    for cp in slab_cps:
        cp.wait()

    def gather_cps(cc, slot):
        return [pltpu.make_async_copy(
                    t2.at[slab.at[pl.ds(k * V_W + cc * CH, CH)]],
                    gbuf[slot].at[pl.ds(k * CH, CH)],
                    gsem[slot]) for k in range(K)]

    def out_cp(cc, slot):
        return pltpu.make_async_copy(
            acc[slot], s_out.at[pl.ds(wbase + cc * CH, CH)], osem[slot])

    def accumulate(slot):
        gb, ab = gbuf[slot], acc[slot]

        def row_body(r, c2):
            for seg in range(C // 16):
                sl = pl.ds(seg * 16, 16)
                v01 = gb[0 * CH + r, sl] + gb[1 * CH + r, sl]
                v23 = gb[2 * CH + r, sl] + gb[3 * CH + r, sl]
                v45 = gb[4 * CH + r, sl] + gb[5 * CH + r, sl]
                ab[r, sl] = (v01 + v23) + v45
            return c2

        lax.fori_loop(0, CH, row_body, 0, unroll=2)

    for cp in gather_cps(0, 0):
        cp.start()

    def outer(c2, carry):
        for s_ in range(2):
            cc = c2 * 2 + s_

            @pl.when(cc < NCH_W)
            def _proc():
                nxt = cc + 1

                @pl.when(nxt < NCH_W)
                def _prefetch():
                    for cp in gather_cps(nxt, 1 - s_):
                        cp.start()

                for cp in gather_cps(cc, s_):
                    cp.wait()

                @pl.when(cc >= 2)
                def _wait_out():
                    out_cp(cc - 2, s_).wait()

                accumulate(s_)
                out_cp(cc, s_).start()

        return carry

    lax.fori_loop(0, (NCH_W + 1) // 2, outer, 0)
    out_cp(NCH_W - 2, (NCH_W - 2) % 2).wait()
    out_cp(NCH_W - 1, (NCH_W - 1) % 2).wait()


def _sc_call(t2, idxr):
    mesh = plsc.VectorSubcoreMesh(core_axis_name="c", subcore_axis_name="s")
    fn = pl.kernel(
        _sc_body,
        mesh=mesh,
        out_type=jax.ShapeDtypeStruct((V_PAD, C), jnp.float32),
        scratch_types=(
            [pltpu.VMEM((NCH_W * K * CH,), jnp.int32)]
            + [pltpu.VMEM((K * CH, C), jnp.float32) for _ in range(2)]
            + [pltpu.VMEM((CH, C), jnp.float32) for _ in range(2)]
            + [pltpu.SemaphoreType.DMA for _ in range(4)]
        ),
    )
    return fn(t2, idxr)


def _norm_body(fe_ref, s_ref, w0_ref, out_ref, sum_scr, sq_scr):
    j = pl.program_id(0)

    @pl.when(j == 0)
    def _init():
        sum_scr[...] = jnp.zeros_like(sum_scr)
        sq_scr[...] = jnp.zeros_like(sq_scr)

    def compute_p():
        pb = lax.dot_general(fe_ref[...], w0_ref[...],
                             (((1,), (1,)), ((), ())),
                             preferred_element_type=jnp.float32)
        return pb + s_ref[...]

    @pl.when(j < NB2)
    def _sweep1():
        pb = compute_p()
        rows = j * VB2 + lax.broadcasted_iota(jnp.int32, (VB2, C), 0)
        pbm = jnp.where(rows < V, pb, 0.0)
        sum_scr[...] += jnp.sum(pbm, axis=0, keepdims=True)
        sq_scr[...] += jnp.sum(pbm * pbm, axis=0, keepdims=True)

    @pl.when(j >= NB2)
    def _sweep2():
        mean = sum_scr[...] / V
        var = sq_scr[...] / V - mean * mean
        scale = lax.rsqrt(var + EPS)
        out_ref[...] = jnp.maximum((compute_p() - mean) * scale, 0.0)


def _norm_call(fet, s, w0):
    return pl.pallas_call(
        _norm_body,
        grid=(2 * NB2,),
        in_specs=[
            pl.BlockSpec((VB2, C),
                         lambda j: (jnp.where(j < NB2, j, j - NB2), 0)),
            pl.BlockSpec((VB2, C),
                         lambda j: (jnp.where(j < NB2, j, j - NB2), 0)),
            pl.BlockSpec((C, C), lambda j: (0, 0)),
        ],
        out_specs=pl.BlockSpec(
            (VB2, C), lambda j: (jnp.where(j < NB2, 0, j - NB2), 0)),
        out_shape=jax.ShapeDtypeStruct((V, C), jnp.float32),
        scratch_shapes=[
            pltpu.VMEM((1, C), jnp.float32),
            pltpu.VMEM((1, C), jnp.float32),
        ],
    )(fet, s, w0)


def kernel(fe, neighbor_idx, W, b):
    del b
    fet = jnp.transpose(fe[0], (1, 0))
    wk = W[:, :, 0, :]
    w0 = wk[:, :, 0]
    wn = jnp.transpose(wk[:, :, 1:], (2, 1, 0))

    idxt = jnp.transpose(neighbor_idx[0], (1, 0)).astype(jnp.int32)
    idxt = idxt + (jnp.arange(K, dtype=jnp.int32) * V)[:, None]
    idxr = jnp.pad(idxt, ((0, 0), (0, V_PAD - V))).reshape(-1)

    t = _mm_call(fet, wn)
    t2 = t.reshape(K * V, C)
    s = _sc_call(t2, idxr)
    out = _norm_call(fet, s, w0)
    out = jnp.transpose(out, (1, 0))[None]
    return out, out


# speedup vs baseline: 1.1066x; 1.1066x over previous
import jax
import jax.numpy as jnp
from jax import lax
from jax.experimental import pallas as pl
from jax.experimental.pallas import tpu as pltpu
from jax.experimental.pallas import tpu_sc as plsc

C = 128
V = 100000
K = 6
CH = 56
NW = 32
V_PAD = ((V + CH * NW - 1) // (CH * NW)) * (CH * NW)
NCHUNK = V_PAD // CH
NCH_W = NCHUNK // NW
V_W = NCH_W * CH

VB1 = 4096
NB1 = (V + VB1 - 1) // VB1
VB2 = 14336
NB2 = V_PAD // VB2
EPS = 1e-5


def _mm_body(fet_ref, wn_ref, t_ref):
    fetb = fet_ref[...]
    for k in range(K):
        t_ref[k] = lax.dot_general(
            fetb, wn_ref[k], (((1,), (0,)), ((), ())),
            preferred_element_type=jnp.float32)


def _mm_call(fet, wn):
    return pl.pallas_call(
        _mm_body,
        grid=(NB1,),
        in_specs=[
            pl.BlockSpec((VB1, C), lambda j: (j, 0)),
            pl.BlockSpec((K, C, C), lambda j: (0, 0, 0)),
        ],
        out_specs=pl.BlockSpec((K, VB1, C), lambda j: (0, j, 0)),
        out_shape=jax.ShapeDtypeStruct((K, V, C), jnp.float32),
    )(fet, wn)


def _sc_body(t2, idxr, s_out, slab, gb0, gb1, ac0, ac1,
             gs0, gs1, os0, os1):
    gbuf = (gb0, gb1)
    acc = (ac0, ac1)
    gsem = (gs0, gs1)
    osem = (os0, os1)
    wid = lax.axis_index("s") * 2 + lax.axis_index("c")
    wbase = wid * V_W

    pltpu.sync_copy(idxr.at[pl.ds(wid * (NCH_W * K * CH), NCH_W * K * CH)],
                    slab)

    def gather_cps(cc, slot):
        return [pltpu.make_async_copy(
                    t2.at[slab.at[pl.ds((cc * K + k) * CH, CH)]],
                    gbuf[slot].at[pl.ds(k * CH, CH)],
                    gsem[slot]) for k in range(K)]

    def out_cp(cc, slot):
        return pltpu.make_async_copy(
            acc[slot], s_out.at[pl.ds(wbase + cc * CH, CH)], osem[slot])

    def accumulate(slot):
        gb, ab = gbuf[slot], acc[slot]

        def row_body(r, c2):
            for seg in range(C // 16):
                sl = pl.ds(seg * 16, 16)
                v01 = gb[0 * CH + r, sl] + gb[1 * CH + r, sl]
                v23 = gb[2 * CH + r, sl] + gb[3 * CH + r, sl]
                v45 = gb[4 * CH + r, sl] + gb[5 * CH + r, sl]
                ab[r, sl] = (v01 + v23) + v45
            return c2

        lax.fori_loop(0, CH, row_body, 0, unroll=2)

    for cp in gather_cps(0, 0):
        cp.start()

    def outer(c2, carry):
        for s_ in range(2):
            cc = c2 * 2 + s_

            @pl.when(cc < NCH_W)
            def _proc():
                nxt = cc + 1

                @pl.when(nxt < NCH_W)
                def _prefetch():
                    for cp in gather_cps(nxt, 1 - s_):
                        cp.start()

                for cp in gather_cps(cc, s_):
                    cp.wait()

                @pl.when(cc >= 2)
                def _wait_out():
                    out_cp(cc - 2, s_).wait()

                accumulate(s_)
                out_cp(cc, s_).start()

        return carry

    lax.fori_loop(0, (NCH_W + 1) // 2, outer, 0)
    out_cp(NCH_W - 2, (NCH_W - 2) % 2).wait()
    out_cp(NCH_W - 1, (NCH_W - 1) % 2).wait()


def _sc_call(t2, idxr):
    mesh = plsc.VectorSubcoreMesh(core_axis_name="c", subcore_axis_name="s")
    fn = pl.kernel(
        _sc_body,
        mesh=mesh,
        out_type=jax.ShapeDtypeStruct((V_PAD, C), jnp.float32),
        scratch_types=(
            [pltpu.VMEM((NCH_W * K * CH,), jnp.int32)]
            + [pltpu.VMEM((K * CH, C), jnp.float32) for _ in range(2)]
            + [pltpu.VMEM((CH, C), jnp.float32) for _ in range(2)]
            + [pltpu.SemaphoreType.DMA for _ in range(4)]
        ),
    )
    return fn(t2, idxr)


def _norm_body(fe_ref, s_ref, w0_ref, out_ref, sum_scr, sq_scr):
    j = pl.program_id(0)

    @pl.when(j == 0)
    def _init():
        sum_scr[...] = jnp.zeros_like(sum_scr)
        sq_scr[...] = jnp.zeros_like(sq_scr)

    def compute_p():
        pb = lax.dot_general(fe_ref[...], w0_ref[...],
                             (((1,), (1,)), ((), ())),
                             preferred_element_type=jnp.float32)
        return pb + s_ref[...]

    @pl.when(j < NB2)
    def _sweep1():
        pb = compute_p()
        rows = j * VB2 + lax.broadcasted_iota(jnp.int32, (VB2, C), 0)
        pbm = jnp.where(rows < V, pb, 0.0)
        sum_scr[...] += jnp.sum(pbm, axis=0, keepdims=True)
        sq_scr[...] += jnp.sum(pbm * pbm, axis=0, keepdims=True)

    @pl.when(j >= NB2)
    def _sweep2():
        mean = sum_scr[...] / V
        var = sq_scr[...] / V - mean * mean
        scale = lax.rsqrt(var + EPS)
        out_ref[...] = jnp.maximum((compute_p() - mean) * scale, 0.0)


def _norm_call(fet, s, w0):
    return pl.pallas_call(
        _norm_body,
        grid=(2 * NB2,),
        in_specs=[
            pl.BlockSpec((VB2, C),
                         lambda j: (jnp.where(j < NB2, j, j - NB2), 0)),
            pl.BlockSpec((VB2, C),
                         lambda j: (jnp.where(j < NB2, j, j - NB2), 0)),
            pl.BlockSpec((C, C), lambda j: (0, 0)),
        ],
        out_specs=pl.BlockSpec(
            (VB2, C), lambda j: (jnp.where(j < NB2, 0, j - NB2), 0)),
        out_shape=jax.ShapeDtypeStruct((V, C), jnp.float32),
        scratch_shapes=[
            pltpu.VMEM((1, C), jnp.float32),
            pltpu.VMEM((1, C), jnp.float32),
        ],
    )(fet, s, w0)


def kernel(fe, neighbor_idx, W, b):
    del b
    fet = jnp.transpose(fe[0], (1, 0))
    wk = W[:, :, 0, :]
    w0 = wk[:, :, 0]
    wn = jnp.transpose(wk[:, :, 1:], (2, 1, 0))

    idxr = jnp.pad(neighbor_idx[0].astype(jnp.int32),
                   ((0, V_PAD - V), (0, 0)))
    idxr = idxr + (jnp.arange(K, dtype=jnp.int32) * V)[None, :]
    idxr = idxr.reshape(NCHUNK, CH, K).transpose(0, 2, 1).reshape(-1)

    t = _mm_call(fet, wn)
    t2 = t.reshape(K * V, C)
    s = _sc_call(t2, idxr)
    out = _norm_call(fet, s, w0)
    out = jnp.transpose(out, (1, 0))[None]
    return out, out
